# baseline (speedup 1.0000x reference)
import functools

import jax
import jax.numpy as jnp
from jax import lax
from jax.experimental import pallas as pl
from jax.experimental.pallas import tpu as pltpu
from jax.experimental.pallas import tpu_sc as plsc

CAP_FACTOR = 0.5
AUX_W = 0.01
EPS = 1e-9

STC = 2560
TC_CHUNK = 1280
SC_TB = 16
SC_WORKERS_PER_B = 8


def _tc_logits_body(h_ref, w_ref, out_ref):
    x = h_ref[0].astype(jnp.bfloat16).astype(jnp.float32)
    w = w_ref[...].astype(jnp.bfloat16).astype(jnp.float32)
    out_ref[0, 0, :] = jnp.sum(x * w, axis=-1)


def _bf16_round(x):
    u = lax.bitcast_convert_type(x, jnp.int32)
    lsb = lax.shift_right_logical(u, 16) & jnp.int32(1)
    u = (u + jnp.int32(0x7FFF) + lsb) & jnp.int32(~0xFFFF)
    return lax.bitcast_convert_type(u, jnp.float32)


def _sc_logits_body(h_ref, w_ref, out_ref, xbuf0, xbuf1, wbuf, obuf, rbuf,
                    sem0, sem1, *, sdim, d, stc, ssc, seg, tb):
    wid = lax.axis_index("s") * 2 + lax.axis_index("c")
    bidx = wid // SC_WORKERS_PER_B
    jidx = wid % SC_WORKERS_PER_B
    row0 = bidx * sdim + stc + jidx * seg
    obase = bidx * ssc + jidx * seg

    pltpu.sync_copy(w_ref, wbuf)

    def _wround(j, carry):
        wv = wbuf[pl.ds(j * 16, 16)]
        wbuf[pl.ds(j * 16, 16)] = _bf16_round(wv)
        return carry

    lax.fori_loop(0, d // 16, _wround, 0)

    bufs = (xbuf0, xbuf1)
    sems = (sem0, sem1)
    nblk = seg // tb

    def _copy(blk):
        slot = blk % 2
        src = h_ref.at[pl.ds(row0 + blk * tb, tb)]
        return pltpu.make_async_copy(src, bufs[slot], sems[slot])

    lane_iota = lax.iota(jnp.int32, 16)

    def _lane_total(x):
        for sh in (8, 4, 2, 1):
            rbuf[...] = x
            idx = jnp.bitwise_and(lane_iota + sh, 15)
            x = x + plsc.load_gather(rbuf, [idx])
        return x

    def _compute_block(buf, blk):
        def gbody(g, sums):
            t0 = g * 4

            def jbody(jj, accs):
                a0, a1, a2, a3 = accs
                for u in range(4):
                    off = jj * 64 + u * 16
                    wv = wbuf[pl.ds(off, 16)]
                    xs = []
                    for t in range(4):
                        xv = _bf16_round(buf[t0 + t, pl.ds(off, 16)])
                        xs.append(xv * wv)
                    a0 += xs[0]
                    a1 += xs[1]
                    a2 += xs[2]
                    a3 += xs[3]
                return (a0, a1, a2, a3)

            z = jnp.zeros((16,), jnp.float32)
            accs = lax.fori_loop(0, d // 64, jbody, (z, z, z, z))
            for t in range(4):
                tot = _lane_total(accs[t])
                sums = jnp.where(lane_iota == t0 + t, tot, sums)
            return sums

        sums = lax.fori_loop(0, tb // 4, gbody, jnp.zeros((16,), jnp.float32))
        obuf[pl.ds(blk * tb, 16)] = sums

    _copy(0).start()
    for blk in range(nblk):
        if blk + 1 < nblk:
            _copy(blk + 1).start()
        _copy(blk).wait()
        _compute_block(bufs[blk % 2], blk)

    pltpu.sync_copy(obuf, out_ref.at[pl.ds(obase, seg)])


def _select_body(lt_ref, ls_ref, b_ref, probs_ref, mask_ref, aux_ref, *, k, s):
    lt = lt_ref[:, 0, :]
    ls = ls_ref[...]
    logits = jnp.concatenate([lt, ls], axis=1) + b_ref[0, 0]
    nb = logits.shape[0]

    ibits = lax.bitcast_convert_type(logits, jnp.int32)
    skey = jnp.where(ibits < 0, ibits ^ jnp.int32(0x7FFFFFFF), ibits)
    ukey = lax.bitcast_convert_type(skey, jnp.uint32) ^ jnp.uint32(0x80000000)

    def _bit_step(t, T):
        cand = T | lax.shift_left(jnp.uint32(1), jnp.uint32(31 - t))
        cnt = jnp.sum((ukey >= cand).astype(jnp.int32), axis=1, keepdims=True)
        return jnp.where(cnt >= k, cand, T)

    T = lax.fori_loop(0, 32, _bit_step, jnp.zeros((nb, 1), jnp.uint32))

    gt = ukey > T
    eq = ukey == T
    r = k - jnp.sum(gt.astype(jnp.int32), axis=1, keepdims=True)
    idx = lax.broadcasted_iota(jnp.int32, logits.shape, 1)

    def _lb_step(_, lo_hi):
        lo, hi = lo_hi
        mid = (lo + hi) // 2
        cnt = jnp.sum((eq & (idx < mid)).astype(jnp.int32), axis=1,
                      keepdims=True)
        take_hi = cnt >= r
        return (jnp.where(take_hi, lo, mid + 1), jnp.where(take_hi, mid, hi))

    _, m = lax.fori_loop(0, 13, _lb_step,
                         (jnp.zeros((nb, 1), jnp.int32),
                          jnp.full((nb, 1), s, jnp.int32)))

    sel = gt | (eq & (idx < m))
    mask_f = sel.astype(jnp.float32)
    probs = jax.nn.sigmoid(logits)
    bce = -(mask_f * jnp.log(probs + EPS)
            + (1.0 - mask_f) * jnp.log(1.0 - probs + EPS))
    aux = AUX_W * jnp.mean(bce, axis=1)
    probs_ref[...] = probs
    mask_ref[...] = mask_f
    aux_ref[...] = jnp.broadcast_to(aux[:, None], aux_ref.shape)


@jax.jit
def kernel(hidden, router_weight, router_bias):
    b, s, d = hidden.shape
    k = int(s * CAP_FACTOR)
    stc = STC
    ssc = s - stc
    seg = ssc // SC_WORKERS_PER_B
    tb = SC_TB

    w1 = router_weight.astype(jnp.float32)
    w2 = w1.reshape(1, d)
    b2 = jnp.asarray(router_bias, jnp.float32).reshape(1, 1)
    h2 = hidden.reshape(b * s, d)

    mesh = plsc.VectorSubcoreMesh(core_axis_name="c", subcore_axis_name="s",
                                  num_cores=2, num_subcores=16)
    sc_logits = pl.kernel(
        functools.partial(_sc_logits_body, sdim=s, d=d, stc=stc, ssc=ssc,
                          seg=seg, tb=tb),
        out_type=jax.ShapeDtypeStruct((b * ssc,), jnp.float32),
        mesh=mesh,
        scratch_types=[
            pltpu.VMEM((tb, d), jnp.float32),
            pltpu.VMEM((tb, d), jnp.float32),
            pltpu.VMEM((d,), jnp.float32),
            pltpu.VMEM((seg,), jnp.float32),
            pltpu.VMEM((16,), jnp.float32),
            pltpu.SemaphoreType.DMA,
            pltpu.SemaphoreType.DMA,
        ],
        compiler_params=pltpu.CompilerParams(needs_layout_passes=False),
    )
    ls = sc_logits(h2, w1).reshape(b, ssc)

    lt = pl.pallas_call(
        _tc_logits_body,
        grid=(b, stc // TC_CHUNK),
        in_specs=[
            pl.BlockSpec((1, TC_CHUNK, d), lambda i, c: (i, c, 0)),
            pl.BlockSpec((1, d), lambda i, c: (0, 0)),
        ],
        out_specs=pl.BlockSpec((1, 1, TC_CHUNK), lambda i, c: (i, 0, c)),
        out_shape=jax.ShapeDtypeStruct((b, 1, stc), jnp.float32),
        compiler_params=pltpu.CompilerParams(
            dimension_semantics=("arbitrary", "arbitrary")),
    )(hidden, w2)

    probs, mask, aux2 = pl.pallas_call(
        functools.partial(_select_body, k=k, s=s),
        out_shape=[
            jax.ShapeDtypeStruct((b, s), jnp.float32),
            jax.ShapeDtypeStruct((b, s), jnp.float32),
            jax.ShapeDtypeStruct((b, 128), jnp.float32),
        ],
    )(lt, ls, b2)

    return probs, mask, aux2[:, 0]


# speedup vs baseline: 1.0486x; 1.0486x over previous
import functools

import jax
import jax.numpy as jnp
from jax import lax
from jax.experimental import pallas as pl
from jax.experimental.pallas import tpu as pltpu
from jax.experimental.pallas import tpu_sc as plsc

CAP_FACTOR = 0.5
AUX_W = 0.01
EPS = 1e-9

STC = 2688
TC_CHUNK = 896
SC_TB = 16
SC_WORKERS_PER_B = 8


def _tc_logits_body(h_ref, w_ref, out_ref):
    x = h_ref[0].astype(jnp.bfloat16).astype(jnp.float32)
    w = w_ref[...].astype(jnp.bfloat16).astype(jnp.float32)
    out_ref[0, 0, :] = jnp.sum(x * w, axis=-1)


def _bf16_round(x):
    u = lax.bitcast_convert_type(x, jnp.int32)
    lsb = lax.shift_right_logical(u, 16) & jnp.int32(1)
    u = (u + jnp.int32(0x7FFF) + lsb) & jnp.int32(~0xFFFF)
    return lax.bitcast_convert_type(u, jnp.float32)


def _sc_logits_body(h_ref, w_ref, out_ref, xbuf0, xbuf1, wbuf, obuf, rbuf,
                    sem0, sem1, *, sdim, d, stc, ssc, seg, tb):
    wid = lax.axis_index("s") * 2 + lax.axis_index("c")
    bidx = wid // SC_WORKERS_PER_B
    jidx = wid % SC_WORKERS_PER_B
    row0 = bidx * sdim + stc + jidx * seg
    obase = bidx * ssc + jidx * seg

    pltpu.sync_copy(w_ref, wbuf)

    def _wround(j, carry):
        wv = wbuf[pl.ds(j * 16, 16)]
        wbuf[pl.ds(j * 16, 16)] = _bf16_round(wv)
        return carry

    lax.fori_loop(0, d // 16, _wround, 0)

    bufs = (xbuf0, xbuf1)
    sems = (sem0, sem1)
    nblk = seg // tb

    def _copy(blk):
        slot = blk % 2
        src = h_ref.at[pl.ds(row0 + blk * tb, tb)]
        return pltpu.make_async_copy(src, bufs[slot], sems[slot])

    lane_iota = lax.iota(jnp.int32, 16)

    def _lane_total(x):
        for sh in (8, 4, 2, 1):
            rbuf[...] = x
            idx = jnp.bitwise_and(lane_iota + sh, 15)
            x = x + plsc.load_gather(rbuf, [idx])
        return x

    def _compute_block(buf, blk):
        def gbody(g, sums):
            t0 = g * 4

            def jbody(jj, accs):
                a0, a1, a2, a3 = accs
                for u in range(4):
                    off = jj * 64 + u * 16
                    wv = wbuf[pl.ds(off, 16)]
                    xs = []
                    for t in range(4):
                        xv = _bf16_round(buf[t0 + t, pl.ds(off, 16)])
                        xs.append(xv * wv)
                    a0 += xs[0]
                    a1 += xs[1]
                    a2 += xs[2]
                    a3 += xs[3]
                return (a0, a1, a2, a3)

            z = jnp.zeros((16,), jnp.float32)
            accs = lax.fori_loop(0, d // 64, jbody, (z, z, z, z))
            for t in range(4):
                tot = _lane_total(accs[t])
                sums = jnp.where(lane_iota == t0 + t, tot, sums)
            return sums

        sums = lax.fori_loop(0, tb // 4, gbody, jnp.zeros((16,), jnp.float32))
        obuf[pl.ds(blk * tb, 16)] = sums

    _copy(0).start()
    for blk in range(nblk):
        if blk + 1 < nblk:
            _copy(blk + 1).start()
        _copy(blk).wait()
        _compute_block(bufs[blk % 2], blk)

    pltpu.sync_copy(obuf, out_ref.at[pl.ds(obase, seg)])


def _select_body(lt_ref, ls_ref, b_ref, probs_ref, mask_ref, aux_ref, *, k, s):
    lt = lt_ref[:, 0, :]
    ls = ls_ref[...]
    logits = jnp.concatenate([lt, ls], axis=1) + b_ref[0, 0]
    nb = logits.shape[0]

    ibits = lax.bitcast_convert_type(logits, jnp.int32)
    skey = jnp.where(ibits < 0, ibits ^ jnp.int32(0x7FFFFFFF), ibits)
    ukey = lax.bitcast_convert_type(skey, jnp.uint32) ^ jnp.uint32(0x80000000)

    def _bit_step(t, T):
        cand = T | lax.shift_left(jnp.uint32(1), jnp.uint32(31 - t))
        cnt = jnp.sum((ukey >= cand).astype(jnp.int32), axis=1, keepdims=True)
        return jnp.where(cnt >= k, cand, T)

    T = lax.fori_loop(0, 32, _bit_step, jnp.zeros((nb, 1), jnp.uint32))

    gt = ukey > T
    eq = ukey == T
    r = k - jnp.sum(gt.astype(jnp.int32), axis=1, keepdims=True)
    idx = lax.broadcasted_iota(jnp.int32, logits.shape, 1)

    def _lb_step(_, lo_hi):
        lo, hi = lo_hi
        mid = (lo + hi) // 2
        cnt = jnp.sum((eq & (idx < mid)).astype(jnp.int32), axis=1,
                      keepdims=True)
        take_hi = cnt >= r
        return (jnp.where(take_hi, lo, mid + 1), jnp.where(take_hi, mid, hi))

    _, m = lax.fori_loop(0, 13, _lb_step,
                         (jnp.zeros((nb, 1), jnp.int32),
                          jnp.full((nb, 1), s, jnp.int32)))

    sel = gt | (eq & (idx < m))
    mask_f = sel.astype(jnp.float32)
    probs = jax.nn.sigmoid(logits)
    bce = -(mask_f * jnp.log(probs + EPS)
            + (1.0 - mask_f) * jnp.log(1.0 - probs + EPS))
    aux = AUX_W * jnp.mean(bce, axis=1)
    probs_ref[...] = probs
    mask_ref[...] = mask_f
    aux_ref[...] = jnp.broadcast_to(aux[:, None], aux_ref.shape)


@jax.jit
def kernel(hidden, router_weight, router_bias):
    b, s, d = hidden.shape
    k = int(s * CAP_FACTOR)
    stc = STC
    ssc = s - stc
    seg = ssc // SC_WORKERS_PER_B
    tb = SC_TB

    w1 = router_weight.astype(jnp.float32)
    w2 = w1.reshape(1, d)
    b2 = jnp.asarray(router_bias, jnp.float32).reshape(1, 1)
    h2 = hidden.reshape(b * s, d)

    mesh = plsc.VectorSubcoreMesh(core_axis_name="c", subcore_axis_name="s",
                                  num_cores=2, num_subcores=16)
    sc_logits = pl.kernel(
        functools.partial(_sc_logits_body, sdim=s, d=d, stc=stc, ssc=ssc,
                          seg=seg, tb=tb),
        out_type=jax.ShapeDtypeStruct((b * ssc,), jnp.float32),
        mesh=mesh,
        scratch_types=[
            pltpu.VMEM((tb, d), jnp.float32),
            pltpu.VMEM((tb, d), jnp.float32),
            pltpu.VMEM((d,), jnp.float32),
            pltpu.VMEM((seg,), jnp.float32),
            pltpu.VMEM((16,), jnp.float32),
            pltpu.SemaphoreType.DMA,
            pltpu.SemaphoreType.DMA,
        ],
        compiler_params=pltpu.CompilerParams(needs_layout_passes=False),
    )
    ls = sc_logits(h2, w1).reshape(b, ssc)

    lt = pl.pallas_call(
        _tc_logits_body,
        grid=(b, stc // TC_CHUNK),
        in_specs=[
            pl.BlockSpec((1, TC_CHUNK, d), lambda i, c: (i, c, 0)),
            pl.BlockSpec((1, d), lambda i, c: (0, 0)),
        ],
        out_specs=pl.BlockSpec((1, 1, TC_CHUNK), lambda i, c: (i, 0, c)),
        out_shape=jax.ShapeDtypeStruct((b, 1, stc), jnp.float32),
        compiler_params=pltpu.CompilerParams(
            dimension_semantics=("arbitrary", "arbitrary")),
    )(hidden, w2)

    probs, mask, aux2 = pl.pallas_call(
        functools.partial(_select_body, k=k, s=s),
        out_shape=[
            jax.ShapeDtypeStruct((b, s), jnp.float32),
            jax.ShapeDtypeStruct((b, s), jnp.float32),
            jax.ShapeDtypeStruct((b, 128), jnp.float32),
        ],
    )(lt, ls, b2)

    return probs, mask, aux2[:, 0]


# speedup vs baseline: 1.1706x; 1.1164x over previous
import functools

import jax
import jax.numpy as jnp
from jax import lax
from jax.experimental import pallas as pl
from jax.experimental.pallas import tpu as pltpu
from jax.experimental.pallas import tpu_sc as plsc

CAP_FACTOR = 0.5
AUX_W = 0.01
EPS = 1e-9

STC = 3072
TC_CHUNK = 1536
SC_TB = 16
SC_WORKERS_PER_B = 8


def _tc_logits_body(h_ref, w_ref, out_ref):
    x = h_ref[0].astype(jnp.bfloat16).astype(jnp.float32)
    w = w_ref[...].astype(jnp.bfloat16).astype(jnp.float32)
    out_ref[0, 0, :] = jnp.sum(x * w, axis=-1)


def _bf16_round(x):
    u = lax.bitcast_convert_type(x, jnp.int32)
    lsb = lax.shift_right_logical(u, 16) & jnp.int32(1)
    u = (u + jnp.int32(0x7FFF) + lsb) & jnp.int32(~0xFFFF)
    return lax.bitcast_convert_type(u, jnp.float32)


def _sc_logits_body(h_ref, w_ref, out_ref, xbuf0, xbuf1, wbuf, obuf, rbuf,
                    sem0, sem1, *, sdim, d, stc, ssc, seg, tb):
    wid = lax.axis_index("s") * 2 + lax.axis_index("c")
    bidx = wid // SC_WORKERS_PER_B
    jidx = wid % SC_WORKERS_PER_B
    row0 = bidx * sdim + stc + jidx * seg
    obase = bidx * ssc + jidx * seg

    pltpu.sync_copy(w_ref, wbuf)

    def _wround(j, carry):
        wv = wbuf[pl.ds(j * 16, 16)]
        wbuf[pl.ds(j * 16, 16)] = _bf16_round(wv)
        return carry

    lax.fori_loop(0, d // 16, _wround, 0)

    bufs = (xbuf0, xbuf1)
    sems = (sem0, sem1)
    nblk = seg // tb

    def _copy(blk):
        slot = blk % 2
        src = h_ref.at[pl.ds(row0 + blk * tb, tb)]
        return pltpu.make_async_copy(src, bufs[slot], sems[slot])

    lane_iota = lax.iota(jnp.int32, 16)

    def _lane_total(x):
        for sh in (8, 4, 2, 1):
            rbuf[...] = x
            idx = jnp.bitwise_and(lane_iota + sh, 15)
            x = x + plsc.load_gather(rbuf, [idx])
        return x

    def _compute_block(buf, blk):
        def gbody(g, sums):
            t0 = g * 4

            def jbody(jj, accs):
                a0, a1, a2, a3 = accs
                for u in range(4):
                    off = jj * 64 + u * 16
                    wv = wbuf[pl.ds(off, 16)]
                    xs = []
                    for t in range(4):
                        xv = _bf16_round(buf[t0 + t, pl.ds(off, 16)])
                        xs.append(xv * wv)
                    a0 += xs[0]
                    a1 += xs[1]
                    a2 += xs[2]
                    a3 += xs[3]
                return (a0, a1, a2, a3)

            z = jnp.zeros((16,), jnp.float32)
            accs = lax.fori_loop(0, d // 64, jbody, (z, z, z, z))
            for t in range(4):
                tot = _lane_total(accs[t])
                sums = jnp.where(lane_iota == t0 + t, tot, sums)
            return sums

        sums = lax.fori_loop(0, tb // 4, gbody, jnp.zeros((16,), jnp.float32))
        obuf[pl.ds(blk * tb, 16)] = sums

    _copy(0).start()
    for blk in range(nblk):
        if blk + 1 < nblk:
            _copy(blk + 1).start()
        _copy(blk).wait()
        _compute_block(bufs[blk % 2], blk)

    pltpu.sync_copy(obuf, out_ref.at[pl.ds(obase, seg)])


def _select_body(lt_ref, ls_ref, b_ref, probs_ref, mask_ref, aux_ref, *, k, s):
    lt = lt_ref[:, 0, :]
    ls = ls_ref[...]
    logits = jnp.concatenate([lt, ls], axis=1) + b_ref[0, 0]
    nb = logits.shape[0]

    ibits = lax.bitcast_convert_type(logits, jnp.int32)
    skey = jnp.where(ibits < 0, ibits ^ jnp.int32(0x7FFFFFFF), ibits)
    ukey = lax.bitcast_convert_type(skey, jnp.uint32) ^ jnp.uint32(0x80000000)

    def _bit_step(t, T):
        cand = T | lax.shift_left(jnp.uint32(1), jnp.uint32(31 - t))
        cnt = jnp.sum((ukey >= cand).astype(jnp.int32), axis=1, keepdims=True)
        return jnp.where(cnt >= k, cand, T)

    T = lax.fori_loop(0, 32, _bit_step, jnp.zeros((nb, 1), jnp.uint32))

    gt = ukey > T
    eq = ukey == T
    r = k - jnp.sum(gt.astype(jnp.int32), axis=1, keepdims=True)
    idx = lax.broadcasted_iota(jnp.int32, logits.shape, 1)

    def _lb_step(_, lo_hi):
        lo, hi = lo_hi
        mid = (lo + hi) // 2
        cnt = jnp.sum((eq & (idx < mid)).astype(jnp.int32), axis=1,
                      keepdims=True)
        take_hi = cnt >= r
        return (jnp.where(take_hi, lo, mid + 1), jnp.where(take_hi, mid, hi))

    _, m = lax.fori_loop(0, 13, _lb_step,
                         (jnp.zeros((nb, 1), jnp.int32),
                          jnp.full((nb, 1), s, jnp.int32)))

    sel = gt | (eq & (idx < m))
    mask_f = sel.astype(jnp.float32)
    probs = jax.nn.sigmoid(logits)
    bce = -(mask_f * jnp.log(probs + EPS)
            + (1.0 - mask_f) * jnp.log(1.0 - probs + EPS))
    aux = AUX_W * jnp.mean(bce, axis=1)
    probs_ref[...] = probs
    mask_ref[...] = mask_f
    aux_ref[...] = jnp.broadcast_to(aux[:, None], aux_ref.shape)


@jax.jit
def kernel(hidden, router_weight, router_bias):
    b, s, d = hidden.shape
    k = int(s * CAP_FACTOR)
    stc = STC
    ssc = s - stc
    seg = ssc // SC_WORKERS_PER_B
    tb = SC_TB

    w1 = router_weight.astype(jnp.float32)
    w2 = w1.reshape(1, d)
    b2 = jnp.asarray(router_bias, jnp.float32).reshape(1, 1)
    h2 = hidden.reshape(b * s, d)

    mesh = plsc.VectorSubcoreMesh(core_axis_name="c", subcore_axis_name="s",
                                  num_cores=2, num_subcores=16)
    sc_logits = pl.kernel(
        functools.partial(_sc_logits_body, sdim=s, d=d, stc=stc, ssc=ssc,
                          seg=seg, tb=tb),
        out_type=jax.ShapeDtypeStruct((b * ssc,), jnp.float32),
        mesh=mesh,
        scratch_types=[
            pltpu.VMEM((tb, d), jnp.float32),
            pltpu.VMEM((tb, d), jnp.float32),
            pltpu.VMEM((d,), jnp.float32),
            pltpu.VMEM((seg,), jnp.float32),
            pltpu.VMEM((16,), jnp.float32),
            pltpu.SemaphoreType.DMA,
            pltpu.SemaphoreType.DMA,
        ],
        compiler_params=pltpu.CompilerParams(needs_layout_passes=False),
    )
    ls = sc_logits(h2, w1).reshape(b, ssc)

    lt = pl.pallas_call(
        _tc_logits_body,
        grid=(b, stc // TC_CHUNK),
        in_specs=[
            pl.BlockSpec((1, TC_CHUNK, d), lambda i, c: (i, c, 0)),
            pl.BlockSpec((1, d), lambda i, c: (0, 0)),
        ],
        out_specs=pl.BlockSpec((1, 1, TC_CHUNK), lambda i, c: (i, 0, c)),
        out_shape=jax.ShapeDtypeStruct((b, 1, stc), jnp.float32),
        compiler_params=pltpu.CompilerParams(
            dimension_semantics=("arbitrary", "arbitrary")),
    )(hidden, w2)

    probs, mask, aux2 = pl.pallas_call(
        functools.partial(_select_body, k=k, s=s),
        out_shape=[
            jax.ShapeDtypeStruct((b, s), jnp.float32),
            jax.ShapeDtypeStruct((b, s), jnp.float32),
            jax.ShapeDtypeStruct((b, 128), jnp.float32),
        ],
    )(lt, ls, b2)

    return probs, mask, aux2[:, 0]


# speedup vs baseline: 1.1891x; 1.0157x over previous
import functools

import jax
import jax.numpy as jnp
from jax import lax
from jax.experimental import pallas as pl
from jax.experimental.pallas import tpu as pltpu
from jax.experimental.pallas import tpu_sc as plsc

CAP_FACTOR = 0.5
AUX_W = 0.01
EPS = 1e-9

STC = 3584
TC_CHUNK = 1792
SC_TB = 16
SC_WORKERS_PER_B = 8


def _tc_logits_body(h_ref, w_ref, out_ref):
    x = h_ref[0].astype(jnp.bfloat16).astype(jnp.float32)
    w = w_ref[...].astype(jnp.bfloat16).astype(jnp.float32)
    out_ref[0, 0, :] = jnp.sum(x * w, axis=-1)


def _bf16_round(x):
    u = lax.bitcast_convert_type(x, jnp.int32)
    lsb = lax.shift_right_logical(u, 16) & jnp.int32(1)
    u = (u + jnp.int32(0x7FFF) + lsb) & jnp.int32(~0xFFFF)
    return lax.bitcast_convert_type(u, jnp.float32)


def _sc_logits_body(h_ref, w_ref, out_ref, xbuf0, xbuf1, wbuf, obuf, rbuf,
                    sem0, sem1, *, sdim, d, stc, ssc, seg, tb):
    wid = lax.axis_index("s") * 2 + lax.axis_index("c")
    bidx = wid // SC_WORKERS_PER_B
    jidx = wid % SC_WORKERS_PER_B
    row0 = bidx * sdim + stc + jidx * seg
    obase = bidx * ssc + jidx * seg

    pltpu.sync_copy(w_ref, wbuf)

    def _wround(j, carry):
        wv = wbuf[pl.ds(j * 16, 16)]
        wbuf[pl.ds(j * 16, 16)] = _bf16_round(wv)
        return carry

    lax.fori_loop(0, d // 16, _wround, 0)

    bufs = (xbuf0, xbuf1)
    sems = (sem0, sem1)
    nblk = seg // tb

    def _copy(blk):
        slot = blk % 2
        src = h_ref.at[pl.ds(row0 + blk * tb, tb)]
        return pltpu.make_async_copy(src, bufs[slot], sems[slot])

    lane_iota = lax.iota(jnp.int32, 16)

    def _lane_total(x):
        for sh in (8, 4, 2, 1):
            rbuf[...] = x
            idx = jnp.bitwise_and(lane_iota + sh, 15)
            x = x + plsc.load_gather(rbuf, [idx])
        return x

    def _compute_block(buf, blk):
        def gbody(g, sums):
            t0 = g * 4

            def jbody(jj, accs):
                a0, a1, a2, a3 = accs
                for u in range(4):
                    off = jj * 64 + u * 16
                    wv = wbuf[pl.ds(off, 16)]
                    xs = []
                    for t in range(4):
                        xv = _bf16_round(buf[t0 + t, pl.ds(off, 16)])
                        xs.append(xv * wv)
                    a0 += xs[0]
                    a1 += xs[1]
                    a2 += xs[2]
                    a3 += xs[3]
                return (a0, a1, a2, a3)

            z = jnp.zeros((16,), jnp.float32)
            accs = lax.fori_loop(0, d // 64, jbody, (z, z, z, z))
            for t in range(4):
                tot = _lane_total(accs[t])
                sums = jnp.where(lane_iota == t0 + t, tot, sums)
            return sums

        sums = lax.fori_loop(0, tb // 4, gbody, jnp.zeros((16,), jnp.float32))
        obuf[pl.ds(blk * tb, 16)] = sums

    _copy(0).start()
    for blk in range(nblk):
        if blk + 1 < nblk:
            _copy(blk + 1).start()
        _copy(blk).wait()
        _compute_block(bufs[blk % 2], blk)

    pltpu.sync_copy(obuf, out_ref.at[pl.ds(obase, seg)])


def _select_body(lt_ref, ls_ref, b_ref, probs_ref, mask_ref, aux_ref, *, k, s):
    lt = lt_ref[:, 0, :]
    ls = ls_ref[...]
    logits = jnp.concatenate([lt, ls], axis=1) + b_ref[0, 0]
    nb = logits.shape[0]

    ibits = lax.bitcast_convert_type(logits, jnp.int32)
    skey = jnp.where(ibits < 0, ibits ^ jnp.int32(0x7FFFFFFF), ibits)
    ukey = lax.bitcast_convert_type(skey, jnp.uint32) ^ jnp.uint32(0x80000000)

    def _bit_step(t, T):
        cand = T | lax.shift_left(jnp.uint32(1), jnp.uint32(31 - t))
        cnt = jnp.sum((ukey >= cand).astype(jnp.int32), axis=1, keepdims=True)
        return jnp.where(cnt >= k, cand, T)

    T = lax.fori_loop(0, 32, _bit_step, jnp.zeros((nb, 1), jnp.uint32))

    gt = ukey > T
    eq = ukey == T
    r = k - jnp.sum(gt.astype(jnp.int32), axis=1, keepdims=True)
    idx = lax.broadcasted_iota(jnp.int32, logits.shape, 1)

    def _lb_step(_, lo_hi):
        lo, hi = lo_hi
        mid = (lo + hi) // 2
        cnt = jnp.sum((eq & (idx < mid)).astype(jnp.int32), axis=1,
                      keepdims=True)
        take_hi = cnt >= r
        return (jnp.where(take_hi, lo, mid + 1), jnp.where(take_hi, mid, hi))

    _, m = lax.fori_loop(0, 13, _lb_step,
                         (jnp.zeros((nb, 1), jnp.int32),
                          jnp.full((nb, 1), s, jnp.int32)))

    sel = gt | (eq & (idx < m))
    mask_f = sel.astype(jnp.float32)
    probs = jax.nn.sigmoid(logits)
    bce = -(mask_f * jnp.log(probs + EPS)
            + (1.0 - mask_f) * jnp.log(1.0 - probs + EPS))
    aux = AUX_W * jnp.mean(bce, axis=1)
    probs_ref[...] = probs
    mask_ref[...] = mask_f
    aux_ref[...] = jnp.broadcast_to(aux[:, None], aux_ref.shape)


@jax.jit
def kernel(hidden, router_weight, router_bias):
    b, s, d = hidden.shape
    k = int(s * CAP_FACTOR)
    stc = STC
    ssc = s - stc
    seg = ssc // SC_WORKERS_PER_B
    tb = SC_TB

    w1 = router_weight.astype(jnp.float32)
    w2 = w1.reshape(1, d)
    b2 = jnp.asarray(router_bias, jnp.float32).reshape(1, 1)
    h2 = hidden.reshape(b * s, d)

    mesh = plsc.VectorSubcoreMesh(core_axis_name="c", subcore_axis_name="s",
                                  num_cores=2, num_subcores=16)
    sc_logits = pl.kernel(
        functools.partial(_sc_logits_body, sdim=s, d=d, stc=stc, ssc=ssc,
                          seg=seg, tb=tb),
        out_type=jax.ShapeDtypeStruct((b * ssc,), jnp.float32),
        mesh=mesh,
        scratch_types=[
            pltpu.VMEM((tb, d), jnp.float32),
            pltpu.VMEM((tb, d), jnp.float32),
            pltpu.VMEM((d,), jnp.float32),
            pltpu.VMEM((seg,), jnp.float32),
            pltpu.VMEM((16,), jnp.float32),
            pltpu.SemaphoreType.DMA,
            pltpu.SemaphoreType.DMA,
        ],
        compiler_params=pltpu.CompilerParams(needs_layout_passes=False),
    )
    ls = sc_logits(h2, w1).reshape(b, ssc)

    lt = pl.pallas_call(
        _tc_logits_body,
        grid=(b, stc // TC_CHUNK),
        in_specs=[
            pl.BlockSpec((1, TC_CHUNK, d), lambda i, c: (i, c, 0)),
            pl.BlockSpec((1, d), lambda i, c: (0, 0)),
        ],
        out_specs=pl.BlockSpec((1, 1, TC_CHUNK), lambda i, c: (i, 0, c)),
        out_shape=jax.ShapeDtypeStruct((b, 1, stc), jnp.float32),
        compiler_params=pltpu.CompilerParams(
            dimension_semantics=("arbitrary", "arbitrary")),
    )(hidden, w2)

    probs, mask, aux2 = pl.pallas_call(
        functools.partial(_select_body, k=k, s=s),
        out_shape=[
            jax.ShapeDtypeStruct((b, s), jnp.float32),
            jax.ShapeDtypeStruct((b, s), jnp.float32),
            jax.ShapeDtypeStruct((b, 128), jnp.float32),
        ],
    )(lt, ls, b2)

    return probs, mask, aux2[:, 0]
